# Initial kernel scaffold; baseline (speedup 1.0000x reference)
import jax
import jax.numpy as jnp
from jax.experimental import pallas as pl


def kernel(ids, mask, table, W_lin, b_lin, W1, b1, W2, b2, W3, b3):
    raise NotImplementedError("write your pallas kernel here")


# speedup vs baseline: 8.7217x; 8.7217x over previous
import jax
import jax.numpy as jnp
from jax.experimental import pallas as pl

B, L = 1024, 20
M, E = 1000, 64
H1, H2 = 256, 128
MP = 1024
BB = 256


def _body(ids_ref, mask_ref, table_ref, tableT_ref, wlin_ref, blin_ref,
          w1_ref, b1_ref, w2_ref, b2_ref, w3_ref, b3_ref, out_ref):
    ids = ids_ref[...]
    mask = mask_ref[...]
    iota = jax.lax.broadcasted_iota(jnp.int32, (BB, MP), 1)
    counts = jnp.zeros((BB, MP), jnp.float32)
    for l in range(L):
        idc = ids[:, l][:, None]
        mc = mask[:, l][:, None]
        counts = counts + jnp.where(iota == idc, mc, 0.0)
    denom = jnp.clip(jnp.sum(mask, axis=1, keepdims=True), 1.0, None)
    f32 = jnp.float32
    user_sum = jnp.dot(counts, table_ref[...], preferred_element_type=f32)
    user = user_sum / denom
    lin = jnp.dot(counts.astype(jnp.bfloat16), wlin_ref[...],
                  preferred_element_type=f32) + blin_ref[...]
    fm = jnp.dot(user, tableT_ref[...], preferred_element_type=f32)
    h = jnp.maximum(jnp.dot(user, w1_ref[...], preferred_element_type=f32)
                    + b1_ref[...], 0.0)
    h = jnp.maximum(jnp.dot(h, w2_ref[...], preferred_element_type=f32)
                    + b2_ref[...], 0.0)
    mlp = jnp.dot(h, w3_ref[...], preferred_element_type=f32) + b3_ref[...]
    out_ref[...] = jax.nn.sigmoid(lin + fm + mlp)


def kernel(ids, mask, table, W_lin, b_lin, W1, b1, W2, b2, W3, b3):
    ids32 = ids.astype(jnp.int32)
    table_pad = jnp.pad(table, ((0, MP - (M + 1)), (0, 0)))
    tableT = table[:M].T
    wlin_pad = jnp.pad(W_lin.astype(jnp.bfloat16),
                       ((0, MP - (M + 1)), (0, 0)))
    full = lambda shape: pl.BlockSpec(shape, lambda i: (0, 0))
    out = pl.pallas_call(
        _body,
        grid=(B // BB,),
        in_specs=[
            pl.BlockSpec((BB, L), lambda i: (i, 0)),
            pl.BlockSpec((BB, L), lambda i: (i, 0)),
            full((MP, E)),
            full((E, M)),
            full((MP, M)),
            full((1, M)),
            full((E, H1)),
            full((1, H1)),
            full((H1, H2)),
            full((1, H2)),
            full((H2, M)),
            full((1, M)),
        ],
        out_specs=pl.BlockSpec((BB, M), lambda i: (i, 0)),
        out_shape=jax.ShapeDtypeStruct((B, M), jnp.float32),
    )(ids32, mask, table_pad, tableT, wlin_pad, b_lin[None, :],
      W1, b1[None, :], W2, b2[None, :], W3, b3[None, :])
    return out


# speedup vs baseline: 8.9413x; 1.0252x over previous
import jax
import jax.numpy as jnp
from jax import lax
from jax.experimental import pallas as pl

B, L = 1024, 20
M, E = 1000, 64
H1, H2 = 256, 128
MP = 1024
BB = 256
CT = 128


def _body(ids_ref, mask_ref, table_ref, wlin_ref, blin_ref,
          w1_ref, b1_ref, w2_ref, b2_ref, w3_ref, b3_ref, out_ref):
    f32 = jnp.float32
    ids = ids_ref[...]
    mask = mask_ref[...]
    idcols = [ids[:, l][:, None] for l in range(L)]
    mcols = [mask[:, l][:, None] for l in range(L)]
    base_iota = lax.broadcasted_iota(jnp.int32, (BB, CT), 1)
    tiles = []
    for t in range(MP // CT):
        iota_t = base_iota + t * CT
        acc = jnp.zeros((BB, CT), f32)
        for l in range(L):
            acc = acc + jnp.where(iota_t == idcols[l], mcols[l], 0.0)
        tiles.append(acc)
    counts = jnp.concatenate(tiles, axis=1)
    denom = jnp.clip(jnp.sum(mask, axis=1, keepdims=True), 1.0, None)

    tab = jnp.concatenate(
        [table_ref[...], jnp.zeros((MP - (M + 1), E), f32)], axis=0)
    user_sum = jnp.dot(counts, tab, preferred_element_type=f32)
    user = user_sum / denom

    wl = jnp.concatenate(
        [wlin_ref[...].astype(jnp.bfloat16),
         jnp.zeros((MP - (M + 1), M), jnp.bfloat16)], axis=0)
    lin = jnp.dot(counts.astype(jnp.bfloat16), wl,
                  preferred_element_type=f32) + blin_ref[...]

    movies = table_ref[0:M, :]
    fm = lax.dot_general(user, movies, (((1,), (1,)), ((), ())),
                         preferred_element_type=f32)

    h = jnp.maximum(jnp.dot(user, w1_ref[...], preferred_element_type=f32)
                    + b1_ref[...], 0.0)
    h = jnp.maximum(jnp.dot(h, w2_ref[...], preferred_element_type=f32)
                    + b2_ref[...], 0.0)
    mlp = jnp.dot(h, w3_ref[...], preferred_element_type=f32) + b3_ref[...]
    out_ref[...] = jax.nn.sigmoid(lin + fm + mlp)


def kernel(ids, mask, table, W_lin, b_lin, W1, b1, W2, b2, W3, b3):
    full = lambda shape: pl.BlockSpec(shape, lambda i: (0, 0))
    out = pl.pallas_call(
        _body,
        grid=(B // BB,),
        in_specs=[
            pl.BlockSpec((BB, L), lambda i: (i, 0)),
            pl.BlockSpec((BB, L), lambda i: (i, 0)),
            full((M + 1, E)),
            full((M + 1, M)),
            full((1, M)),
            full((E, H1)),
            full((1, H1)),
            full((H1, H2)),
            full((1, H2)),
            full((H2, M)),
            full((1, M)),
        ],
        out_specs=pl.BlockSpec((BB, M), lambda i: (i, 0)),
        out_shape=jax.ShapeDtypeStruct((B, M), jnp.float32),
    )(ids.astype(jnp.int32), mask, table, W_lin, b_lin[None, :],
      W1, b1[None, :], W2, b2[None, :], W3, b3[None, :])
    return out


# speedup vs baseline: 20.7230x; 2.3177x over previous
import jax
import jax.numpy as jnp
from jax import lax
from jax.experimental import pallas as pl
from jax.experimental.pallas import tpu as pltpu

B, L = 1024, 20
M, E = 1000, 64
H1, H2 = 256, 128
MP = 1024
BB = 256

_DN0 = (((0,), (0,)), ((), ()))


def _body(idsT_ref, maskT_ref, tableT_ref, wlinT_ref, blin_ref,
          w1_ref, b1_ref, w2_ref, b2_ref, w3T_ref, b3_ref, out_ref):
    f32 = jnp.float32
    idsT = idsT_ref[...]
    maskT = maskT_ref[...]
    iota = lax.broadcasted_iota(jnp.int32, (MP, BB), 0)
    counts = jnp.zeros((MP, BB), f32)
    for l in range(L):
        idr = idsT[l][None, :]
        mr = maskT[l][None, :]
        counts = counts + jnp.where(iota == idr, mr, 0.0)
    denom = jnp.clip(jnp.sum(maskT, axis=0, keepdims=True), 1.0, None)

    tabT = jnp.concatenate(
        [tableT_ref[...], jnp.zeros((E, MP - (M + 1)), f32)], axis=1)
    user_sum = jnp.dot(tabT, counts, preferred_element_type=f32)
    user = user_sum / denom

    wlT = jnp.concatenate(
        [wlinT_ref[...].astype(jnp.bfloat16),
         jnp.zeros((M, MP - (M + 1)), jnp.bfloat16)], axis=1)
    lin = jnp.dot(wlT, counts.astype(jnp.bfloat16),
                  preferred_element_type=f32) + blin_ref[...][:, None]

    moviesT = tableT_ref[:, 0:M]
    fm = lax.dot_general(moviesT, user, _DN0,
                         preferred_element_type=f32)

    h = jnp.maximum(
        lax.dot_general(w1_ref[...], user, _DN0, preferred_element_type=f32)
        + b1_ref[...][:, None], 0.0)
    h = jnp.maximum(
        lax.dot_general(w2_ref[...], h, _DN0, preferred_element_type=f32)
        + b2_ref[...][:, None], 0.0)
    mlp = jnp.dot(w3T_ref[...], h,
                  preferred_element_type=f32) + b3_ref[...][:, None]
    out_ref[...] = jax.nn.sigmoid(lin + fm + mlp)


def kernel(ids, mask, table, W_lin, b_lin, W1, b1, W2, b2, W3, b3):
    full = lambda shape: pl.BlockSpec(shape, lambda i: tuple(0 for _ in shape))
    args = (ids.astype(jnp.int32).T, mask.T, table.T, W_lin.T, b_lin,
            W1, b1, W2, b2, W3.T, b3)
    args = tuple(pltpu.with_memory_space_constraint(x, pltpu.MemorySpace.HBM)
                 for x in args)
    outT = pl.pallas_call(
        _body,
        grid=(B // BB,),
        in_specs=[
            pl.BlockSpec((L, BB), lambda i: (0, i)),
            pl.BlockSpec((L, BB), lambda i: (0, i)),
            full((E, M + 1)),
            full((M, M + 1)),
            full((M,)),
            full((E, H1)),
            full((H1,)),
            full((H1, H2)),
            full((H2,)),
            full((M, H2)),
            full((M,)),
        ],
        out_specs=pl.BlockSpec((M, BB), lambda i: (0, i)),
        out_shape=jax.ShapeDtypeStruct((M, B), jnp.float32),
    )(*args)
    return outT.T


# speedup vs baseline: 26.9338x; 1.2997x over previous
import jax
import jax.numpy as jnp
from jax import lax
from jax.experimental import pallas as pl
from jax.experimental.pallas import tpu as pltpu

B, L = 1024, 20
M, E = 1000, 64
H1, H2 = 256, 128
MP = 1024
BB = 256

_DN0 = (((0,), (0,)), ((), ()))


def _body(idsT_ref, maskT_ref, tableT_ref, wlinT_ref, blin_ref,
          w1_ref, b1_ref, w2_ref, b2_ref, w3T_ref, b3_ref, out_ref):
    f32 = jnp.float32
    bf16 = jnp.bfloat16
    idsT = idsT_ref[...].astype(jnp.int16)
    maskT = maskT_ref[...]
    iota = lax.broadcasted_iota(jnp.int32, (MP, BB), 0).astype(jnp.int16)
    mbf = maskT.astype(bf16)
    counts = jnp.zeros((MP, BB), bf16)
    for l in range(L):
        idr = idsT[l][None, :]
        mr = mbf[l][None, :]
        counts = counts + jnp.where(iota == idr, mr, jnp.zeros((), bf16))
    denom = jnp.clip(jnp.sum(maskT, axis=0, keepdims=True), 1.0, None)

    tabT = jnp.concatenate(
        [tableT_ref[...].astype(bf16), jnp.zeros((E, MP - (M + 1)), bf16)],
        axis=1)
    user_sum = jnp.dot(tabT, counts, preferred_element_type=f32)
    user = user_sum / denom

    wlT = jnp.concatenate(
        [wlinT_ref[...].astype(jnp.bfloat16),
         jnp.zeros((M, MP - (M + 1)), jnp.bfloat16)], axis=1)
    lin = jnp.dot(wlT, counts,
                  preferred_element_type=f32) + blin_ref[...][:, None]

    moviesT = tableT_ref[:, 0:M]
    fm = lax.dot_general(moviesT, user, _DN0,
                         preferred_element_type=f32)

    h = jnp.maximum(
        lax.dot_general(w1_ref[...], user, _DN0, preferred_element_type=f32)
        + b1_ref[...][:, None], 0.0)
    h = jnp.maximum(
        lax.dot_general(w2_ref[...], h, _DN0, preferred_element_type=f32)
        + b2_ref[...][:, None], 0.0)
    mlp = jnp.dot(w3T_ref[...], h,
                  preferred_element_type=f32) + b3_ref[...][:, None]
    out_ref[...] = jax.nn.sigmoid(lin + fm + mlp)


def kernel(ids, mask, table, W_lin, b_lin, W1, b1, W2, b2, W3, b3):
    full = lambda shape: pl.BlockSpec(shape, lambda i: tuple(0 for _ in shape))
    args = (ids.astype(jnp.int32).T, mask.T, table.T, W_lin.T, b_lin,
            W1, b1, W2, b2, W3.T, b3)
    args = tuple(pltpu.with_memory_space_constraint(x, pltpu.MemorySpace.HBM)
                 for x in args)
    outT = pl.pallas_call(
        _body,
        grid=(B // BB,),
        in_specs=[
            pl.BlockSpec((L, BB), lambda i: (0, i)),
            pl.BlockSpec((L, BB), lambda i: (0, i)),
            full((E, M + 1)),
            full((M, M + 1)),
            full((M,)),
            full((E, H1)),
            full((H1,)),
            full((H1, H2)),
            full((H2,)),
            full((M, H2)),
            full((M,)),
        ],
        out_specs=pl.BlockSpec((M, BB), lambda i: (0, i)),
        out_shape=jax.ShapeDtypeStruct((M, B), jnp.float32),
    )(*args)
    return outT.T
